# Initial kernel scaffold; baseline (speedup 1.0000x reference)
import jax
import jax.numpy as jnp
from jax.experimental import pallas as pl


def kernel(features, edge_index, W0, W1, W2):
    raise NotImplementedError("write your pallas kernel here")


# speedup vs baseline: 2.3016x; 2.3016x over previous
import functools

import jax
import jax.numpy as jnp
from jax import lax
from jax.experimental import pallas as pl
from jax.experimental.pallas import tpu as pltpu
from jax.experimental.pallas import tpu_sc as plsc

N = 10000
E = 320000
D_IN = 128
D_HID = 128
D_OUT = 64

NC = 2
NS = 16
NW = NC * NS
CH = 128
CPT = 80
E_PAD = NW * CPT * CH
N_TAB = 10240
RPT = N_TAB // NS
DEG_W = 16
D = 128

_f32 = jnp.float32
_i32 = jnp.int32


@functools.lru_cache(maxsize=None)
def _mesh():
    return plsc.VectorSubcoreMesh(core_axis_name="c", subcore_axis_name="s",
                                  num_cores=NC, num_subcores=NS)


def _fill(buf, rows, width, value):
    vec = jnp.full((16,), value, _f32)

    def body(i, carry):
        for j in range(width // 16):
            buf[i, pl.ds(j * 16, 16)] = vec
        return carry

    lax.fori_loop(0, rows, body, 0)


@functools.lru_cache(maxsize=None)
def _make_agg():
    @functools.partial(
        pl.kernel,
        out_type=jax.ShapeDtypeStruct((NC, N_TAB, D), _f32),
        mesh=_mesh(),
        scratch_types=[
            pltpu.VMEM((CPT, CH), _i32),
            pltpu.VMEM((2, CH), _i32),
            pltpu.VMEM((CH, D), _f32),
            pltpu.VMEM((CH, D), _f32),
            pltpu.SemaphoreType.DMA,
            pltpu.SemaphoreType.DMA,
            pltpu.SemaphoreType.DMA,
            pltpu.SemaphoreType.DMA,
            pltpu.VMEM_SHARED((N_TAB, D), _f32),
        ],
    )
    def agg(y_hbm, src_hbm, dst_hbm, out_hbm, srcv, dstv, bufa, bufb,
            sema, semb, semd0, semd1, acc):
        cid = lax.axis_index("c")
        sid = lax.axis_index("s")
        wid = sid * NC + cid
        pltpu.sync_copy(src_hbm.at[wid], srcv)
        _fill(bufa, CH, D, 0.0)
        for r in range(RPT // CH):
            pltpu.sync_copy(bufa, acc.at[pl.ds(sid * RPT + r * CH, CH)])
        plsc.subcore_barrier()

        pltpu.async_copy(y_hbm.at[srcv.at[0]], bufa, sema)
        pltpu.async_copy(y_hbm.at[srcv.at[1]], bufb, semb)
        pltpu.async_copy(dst_hbm.at[wid, 0], dstv.at[0], semd0)
        pltpu.async_copy(dst_hbm.at[wid, 1], dstv.at[1], semd1)

        def half(g, buf, sem, slot, semd):
            pltpu.make_async_copy(y_hbm.at[srcv.at[g]], buf, sem).wait()
            pltpu.make_async_copy(dst_hbm.at[wid, g], dstv.at[slot], semd).wait()
            pltpu.sync_copy(buf, acc.at[dstv.at[slot]], add=True)

            @pl.when(g + 2 < CPT)
            def _():
                pltpu.async_copy(y_hbm.at[srcv.at[g + 2]], buf, sem)
                pltpu.async_copy(dst_hbm.at[wid, g + 2], dstv.at[slot], semd)

        def step(k, carry):
            g = k * 2
            half(g, bufa, sema, 0, semd0)
            half(g + 1, bufb, semb, 1, semd1)
            return carry

        lax.fori_loop(0, CPT // 2, step, 0)
        plsc.subcore_barrier()
        pltpu.sync_copy(acc.at[pl.ds(sid * RPT, RPT)],
                        out_hbm.at[cid, pl.ds(sid * RPT, RPT)])

    return agg


BLK = 512


def _scales(deg_ref):
    return lax.rsqrt(jnp.maximum(deg_ref[...], 1.0))


def _tc_first_body(x_ref, od_ref, w_ref, y_ref):
    s = _scales(od_ref)
    y_ref[...] = jnp.dot(x_ref[...] * s, w_ref[...],
                         preferred_element_type=_f32)


def _tc_mid_body(p_ref, od_ref, id_ref, w_ref, y_ref):
    s = _scales(od_ref)
    t = _scales(id_ref)
    h = jnp.maximum((p_ref[0] + p_ref[1]) * t, 0.0) * s
    y_ref[...] = jnp.dot(h, w_ref[...], preferred_element_type=_f32)


def _tc_last_body(p_ref, id_ref, y_ref):
    t = _scales(id_ref)
    y_ref[...] = (p_ref[0][:, :D_OUT] + p_ref[1][:, :D_OUT]) * t


def _deg_spec():
    return pl.BlockSpec((BLK, 1), lambda i: (i, 0))


def _tc_first(x, od, w):
    return pl.pallas_call(
        _tc_first_body,
        grid=(N_TAB // BLK,),
        in_specs=[
            pl.BlockSpec((BLK, D_IN), lambda i: (i, 0)),
            _deg_spec(),
            pl.BlockSpec((D_IN, D), lambda i: (0, 0)),
        ],
        out_specs=pl.BlockSpec((BLK, D), lambda i: (i, 0)),
        out_shape=jax.ShapeDtypeStruct((N_TAB, D), _f32),
    )(x, od, w)


def _tc_mid(p, od, idp, w):
    return pl.pallas_call(
        _tc_mid_body,
        grid=(N_TAB // BLK,),
        in_specs=[
            pl.BlockSpec((NC, BLK, D), lambda i: (0, i, 0)),
            _deg_spec(),
            _deg_spec(),
            pl.BlockSpec((D, D), lambda i: (0, 0)),
        ],
        out_specs=pl.BlockSpec((BLK, D), lambda i: (i, 0)),
        out_shape=jax.ShapeDtypeStruct((N_TAB, D), _f32),
    )(p, od, idp, w)


def _tc_last(p, idp):
    return pl.pallas_call(
        _tc_last_body,
        grid=(N_TAB // BLK,),
        in_specs=[
            pl.BlockSpec((NC, BLK, D), lambda i: (0, i, 0)),
            _deg_spec(),
        ],
        out_specs=pl.BlockSpec((BLK, D_OUT), lambda i: (i, 0)),
        out_shape=jax.ShapeDtypeStruct((N_TAB, D_OUT), _f32),
    )(p, idp)


def kernel(features, edge_index, W0, W1, W2):
    src = edge_index[0].astype(_i32)
    dst = edge_index[1].astype(_i32)
    src3 = jnp.pad(src, (0, E_PAD - E), constant_values=N).reshape(NW, CPT, CH)
    dst3 = jnp.pad(dst, (0, E_PAD - E), constant_values=N).reshape(NW, CPT, CH)
    x = jnp.pad(features, ((0, N_TAB - N), (0, 0)))
    w2p = jnp.pad(W2, ((0, 0), (0, D - D_OUT)))

    agg = _make_agg()
    u = jnp.zeros((N_TAB, D), _f32).at[:N, 0].set(1.0)
    idr = agg(u, src3, dst3)
    odr = agg(u, dst3, src3)
    od = (odr[0, :, 0] + odr[1, :, 0])[:, None]
    idp = (idr[0, :, 0] + idr[1, :, 0])[:, None]

    y0 = _tc_first(x, od, W0)
    p0 = agg(y0, src3, dst3)
    y1 = _tc_mid(p0, od, idp, W1)
    p1 = agg(y1, src3, dst3)
    y2 = _tc_mid(p1, od, idp, w2p)
    p2 = agg(y2, src3, dst3)
    out = _tc_last(p2, idp)
    return out[:N]


# speedup vs baseline: 3.3950x; 1.4751x over previous
import functools

import jax
import jax.numpy as jnp
from jax import lax
from jax.experimental import pallas as pl
from jax.experimental.pallas import tpu as pltpu
from jax.experimental.pallas import tpu_sc as plsc

N = 10000
E = 320000
D_IN = 128
D_HID = 128
D_OUT = 64

NC = 2
NS = 16
NW = NC * NS
CH = 128
CPT = 80
E_PAD = NW * CPT * CH
N_TAB = 10240
RPT = N_TAB // NS
DEG_W = 16
D = 128

_f32 = jnp.float32
_i32 = jnp.int32


@functools.lru_cache(maxsize=None)
def _mesh():
    return plsc.VectorSubcoreMesh(core_axis_name="c", subcore_axis_name="s",
                                  num_cores=NC, num_subcores=NS)


def _fill(buf, rows, width, value):
    vec = jnp.full((16,), value, _f32)

    def body(i, carry):
        for j in range(width // 16):
            buf[i, pl.ds(j * 16, 16)] = vec
        return carry

    lax.fori_loop(0, rows, body, 0)


@functools.lru_cache(maxsize=None)
def _make_agg():
    @functools.partial(
        pl.kernel,
        out_type=jax.ShapeDtypeStruct((NC, N_TAB, D), _f32),
        mesh=_mesh(),
        scratch_types=[
            pltpu.VMEM((CPT, CH), _i32),
            pltpu.VMEM((2, CH), _i32),
            pltpu.VMEM((CH, D), _f32),
            pltpu.VMEM((CH, D), _f32),
            pltpu.SemaphoreType.DMA,
            pltpu.SemaphoreType.DMA,
            pltpu.SemaphoreType.DMA,
            pltpu.SemaphoreType.DMA,
            pltpu.VMEM_SHARED((N_TAB, D), _f32),
        ],
    )
    def agg(y_hbm, src_hbm, dst_hbm, out_hbm, srcv, dstv, bufa, bufb,
            sema, semb, semd0, semd1, acc):
        cid = lax.axis_index("c")
        sid = lax.axis_index("s")
        wid = sid * NC + cid
        pltpu.sync_copy(src_hbm.at[wid], srcv)
        _fill(bufa, CH, D, 0.0)
        for r in range(RPT // CH):
            pltpu.sync_copy(bufa, acc.at[pl.ds(sid * RPT + r * CH, CH)])
        plsc.subcore_barrier()

        pltpu.async_copy(y_hbm.at[srcv.at[0]], bufa, sema)
        pltpu.async_copy(y_hbm.at[srcv.at[1]], bufb, semb)
        pltpu.async_copy(dst_hbm.at[wid, 0], dstv.at[0], semd0)
        pltpu.async_copy(dst_hbm.at[wid, 1], dstv.at[1], semd1)

        def half(g, buf, sem, slot, semd):
            pltpu.make_async_copy(y_hbm.at[srcv.at[g]], buf, sem).wait()
            pltpu.make_async_copy(dst_hbm.at[wid, g], dstv.at[slot], semd).wait()
            pltpu.sync_copy(buf, acc.at[dstv.at[slot]], add=True)

            @pl.when(g + 2 < CPT)
            def _():
                pltpu.async_copy(y_hbm.at[srcv.at[g + 2]], buf, sem)
                pltpu.async_copy(dst_hbm.at[wid, g + 2], dstv.at[slot], semd)

        def step(k, carry):
            g = k * 2
            half(g, bufa, sema, 0, semd0)
            half(g + 1, bufb, semb, 1, semd1)
            return carry

        lax.fori_loop(0, CPT // 2, step, 0)
        plsc.subcore_barrier()
        pltpu.sync_copy(acc.at[pl.ds(sid * RPT, RPT)],
                        out_hbm.at[cid, pl.ds(sid * RPT, RPT)])

    return agg


@functools.lru_cache(maxsize=None)
def _make_degrees():
    @functools.partial(
        pl.kernel,
        out_type=(
            jax.ShapeDtypeStruct((NC, N_TAB, D), _f32),
            jax.ShapeDtypeStruct((NC, N_TAB, D), _f32),
        ),
        mesh=_mesh(),
        scratch_types=[
            pltpu.VMEM((CPT, CH), _i32),
            pltpu.VMEM((CPT, CH), _i32),
            pltpu.VMEM((CH, D), _f32),
            pltpu.VMEM_SHARED((N_TAB, D), _f32),
        ],
    )
    def degk(src_hbm, dst_hbm, od_out, id_out, srcv, dstv, buf, acc):
        cid = lax.axis_index("c")
        sid = lax.axis_index("s")
        wid = sid * NC + cid
        pltpu.sync_copy(src_hbm.at[wid], srcv)
        pltpu.sync_copy(dst_hbm.at[wid], dstv)
        for idx, out in ((srcv, od_out), (dstv, id_out)):
            _fill(buf, CH, D, 0.0)
            for r in range(RPT // CH):
                pltpu.sync_copy(buf, acc.at[pl.ds(sid * RPT + r * CH, CH)])
            _fill(buf, CH, D, 1.0)
            plsc.subcore_barrier()

            def step(g, carry, _idx=idx):
                pltpu.sync_copy(buf, acc.at[_idx.at[g]], add=True)
                return carry

            lax.fori_loop(0, CPT, step, 0)
            plsc.subcore_barrier()
            pltpu.sync_copy(acc.at[pl.ds(sid * RPT, RPT)],
                            out.at[cid, pl.ds(sid * RPT, RPT)])
            plsc.subcore_barrier()

    return degk


BLK = 512


def _scales(deg_ref):
    return lax.rsqrt(jnp.maximum(deg_ref[...], 1.0))


def _tc_first_body(x_ref, od_ref, w_ref, y_ref):
    s = _scales(od_ref)
    y_ref[...] = jnp.dot(x_ref[...] * s, w_ref[...],
                         preferred_element_type=_f32)


def _tc_mid_body(p_ref, od_ref, id_ref, w_ref, y_ref):
    s = _scales(od_ref)
    t = _scales(id_ref)
    h = jnp.maximum((p_ref[0] + p_ref[1]) * t, 0.0) * s
    y_ref[...] = jnp.dot(h, w_ref[...], preferred_element_type=_f32)


def _tc_last_body(p_ref, id_ref, y_ref):
    t = _scales(id_ref)
    y_ref[...] = (p_ref[0][:, :D_OUT] + p_ref[1][:, :D_OUT]) * t


def _deg_spec():
    return pl.BlockSpec((BLK, 1), lambda i: (i, 0))


def _tc_first(x, od, w):
    return pl.pallas_call(
        _tc_first_body,
        grid=(N_TAB // BLK,),
        in_specs=[
            pl.BlockSpec((BLK, D_IN), lambda i: (i, 0)),
            _deg_spec(),
            pl.BlockSpec((D_IN, D), lambda i: (0, 0)),
        ],
        out_specs=pl.BlockSpec((BLK, D), lambda i: (i, 0)),
        out_shape=jax.ShapeDtypeStruct((N_TAB, D), _f32),
    )(x, od, w)


def _tc_mid(p, od, idp, w):
    return pl.pallas_call(
        _tc_mid_body,
        grid=(N_TAB // BLK,),
        in_specs=[
            pl.BlockSpec((NC, BLK, D), lambda i: (0, i, 0)),
            _deg_spec(),
            _deg_spec(),
            pl.BlockSpec((D, D), lambda i: (0, 0)),
        ],
        out_specs=pl.BlockSpec((BLK, D), lambda i: (i, 0)),
        out_shape=jax.ShapeDtypeStruct((N_TAB, D), _f32),
    )(p, od, idp, w)


def _tc_last(p, idp):
    return pl.pallas_call(
        _tc_last_body,
        grid=(N_TAB // BLK,),
        in_specs=[
            pl.BlockSpec((NC, BLK, D), lambda i: (0, i, 0)),
            _deg_spec(),
        ],
        out_specs=pl.BlockSpec((BLK, D_OUT), lambda i: (i, 0)),
        out_shape=jax.ShapeDtypeStruct((N_TAB, D_OUT), _f32),
    )(p, idp)


def kernel(features, edge_index, W0, W1, W2):
    src = edge_index[0].astype(_i32)
    dst = edge_index[1].astype(_i32)
    src3 = jnp.pad(src, (0, E_PAD - E), constant_values=N).reshape(NW, CPT, CH)
    dst3 = jnp.pad(dst, (0, E_PAD - E), constant_values=N).reshape(NW, CPT, CH)
    x = jnp.pad(features, ((0, N_TAB - N), (0, 0)))
    w2p = jnp.pad(W2, ((0, 0), (0, D - D_OUT)))

    agg = _make_agg()
    odr, idr = _make_degrees()(src3, dst3)
    od = (odr[0, :, 0] + odr[1, :, 0])[:, None]
    idp = (idr[0, :, 0] + idr[1, :, 0])[:, None]

    y0 = _tc_first(x, od, W0)
    p0 = agg(y0, src3, dst3)
    y1 = _tc_mid(p0, od, idp, W1)
    p1 = agg(y1, src3, dst3)
    y2 = _tc_mid(p1, od, idp, w2p)
    p2 = agg(y2, src3, dst3)
    out = _tc_last(p2, idp)
    return out[:N]


# speedup vs baseline: 3.3953x; 1.0001x over previous
import functools

import jax
import jax.numpy as jnp
from jax import lax
from jax.experimental import pallas as pl
from jax.experimental.pallas import tpu as pltpu
from jax.experimental.pallas import tpu_sc as plsc

N = 10000
E = 320000
D_IN = 128
D_HID = 128
D_OUT = 64

NC = 2
NS = 16
NW = NC * NS
CH = 128
CPT = 80
E_PAD = NW * CPT * CH
N_TAB = 10240
RPT = N_TAB // NS
D = 128

_f32 = jnp.float32
_i32 = jnp.int32


@functools.lru_cache(maxsize=None)
def _mesh():
    return plsc.VectorSubcoreMesh(core_axis_name="c", subcore_axis_name="s",
                                  num_cores=NC, num_subcores=NS)


def _fill(buf, rows, width, value):
    vec = jnp.full((16,), value, _f32)

    def body(i, carry):
        for j in range(width // 16):
            buf[i, pl.ds(j * 16, 16)] = vec
        return carry

    lax.fori_loop(0, rows, body, 0)


@functools.lru_cache(maxsize=None)
def _make_agg():
    @functools.partial(
        pl.kernel,
        out_type=jax.ShapeDtypeStruct((NC, N_TAB, D), _f32),
        mesh=_mesh(),
        scratch_types=[
            pltpu.VMEM((CPT, CH), _i32),
            pltpu.VMEM((2, CH), _i32),
            pltpu.VMEM((CH, D), _f32),
            pltpu.VMEM((CH, D), _f32),
            pltpu.SemaphoreType.DMA,
            pltpu.SemaphoreType.DMA,
            pltpu.SemaphoreType.DMA,
            pltpu.SemaphoreType.DMA,
            pltpu.VMEM_SHARED((N_TAB, D), _f32),
        ],
    )
    def agg(y_hbm, src_hbm, dst_hbm, out_hbm, srcv, dstv, bufa, bufb,
            sema, semb, semd0, semd1, acc):
        cid = lax.axis_index("c")
        sid = lax.axis_index("s")
        wid = sid * NC + cid
        pltpu.sync_copy(src_hbm.at[wid], srcv)
        _fill(bufa, CH, D, 0.0)
        for r in range(RPT // CH):
            pltpu.sync_copy(bufa, acc.at[pl.ds(sid * RPT + r * CH, CH)])
        plsc.subcore_barrier()

        pltpu.async_copy(y_hbm.at[srcv.at[0]], bufa, sema)
        pltpu.async_copy(y_hbm.at[srcv.at[1]], bufb, semb)
        pltpu.async_copy(dst_hbm.at[wid, 0], dstv.at[0], semd0)
        pltpu.async_copy(dst_hbm.at[wid, 1], dstv.at[1], semd1)

        def half(g, buf, sem, slot, semd):
            pltpu.make_async_copy(y_hbm.at[srcv.at[g]], buf, sem).wait()
            pltpu.make_async_copy(dst_hbm.at[wid, g], dstv.at[slot], semd).wait()
            pltpu.sync_copy(buf, acc.at[dstv.at[slot]], add=True)

            @pl.when(g + 2 < CPT)
            def _():
                pltpu.async_copy(y_hbm.at[srcv.at[g + 2]], buf, sem)
                pltpu.async_copy(dst_hbm.at[wid, g + 2], dstv.at[slot], semd)

        def step(k, carry):
            g = k * 2
            half(g, bufa, sema, 0, semd0)
            half(g + 1, bufb, semb, 1, semd1)
            return carry

        lax.fori_loop(0, CPT // 2, step, 0)
        plsc.subcore_barrier()
        pltpu.sync_copy(acc.at[pl.ds(sid * RPT, RPT)],
                        out_hbm.at[cid, pl.ds(sid * RPT, RPT)])

    return agg


@functools.lru_cache(maxsize=None)
def _make_degrees():
    @functools.partial(
        pl.kernel,
        out_type=(
            jax.ShapeDtypeStruct((NC, N_TAB, D), _f32),
            jax.ShapeDtypeStruct((NC, N_TAB, D), _f32),
        ),
        mesh=_mesh(),
        scratch_types=[
            pltpu.VMEM((CPT, CH), _i32),
            pltpu.VMEM((CPT, CH), _i32),
            pltpu.VMEM((CH, D), _f32),
            pltpu.VMEM_SHARED((N_TAB, D), _f32),
        ],
    )
    def degk(src_hbm, dst_hbm, od_out, id_out, srcv, dstv, buf, acc):
        cid = lax.axis_index("c")
        sid = lax.axis_index("s")
        wid = sid * NC + cid
        pltpu.sync_copy(src_hbm.at[wid], srcv)
        pltpu.sync_copy(dst_hbm.at[wid], dstv)
        for idx, out in ((srcv, od_out), (dstv, id_out)):
            _fill(buf, CH, D, 0.0)
            for r in range(RPT // CH):
                pltpu.sync_copy(buf, acc.at[pl.ds(sid * RPT + r * CH, CH)])
            _fill(buf, CH, D, 1.0)
            plsc.subcore_barrier()

            def step(g, carry, _idx=idx):
                pltpu.sync_copy(buf, acc.at[_idx.at[g]], add=True)
                return carry

            lax.fori_loop(0, CPT, step, 0)
            plsc.subcore_barrier()
            pltpu.sync_copy(acc.at[pl.ds(sid * RPT, RPT)],
                            out.at[cid, pl.ds(sid * RPT, RPT)])
            plsc.subcore_barrier()

    return degk


BLK = 512


def _scales(deg_ref):
    return lax.rsqrt(jnp.maximum(deg_ref[...], 1.0))


def _tc_first_body(x_ref, od_ref, w_ref, y_ref):
    s = _scales(od_ref)
    y_ref[...] = jnp.dot(x_ref[...] * s, w_ref[...],
                         preferred_element_type=_f32)


def _tc_mid_body(p_ref, od_ref, id_ref, w_ref, y_ref):
    s = _scales(od_ref)
    t = _scales(id_ref)
    h = jnp.maximum((p_ref[0] + p_ref[1]) * t, 0.0) * s
    y_ref[...] = jnp.dot(h, w_ref[...], preferred_element_type=_f32)


def _tc_last_body(p_ref, id_ref, y_ref):
    t = _scales(id_ref)
    y_ref[...] = (p_ref[0][:, :D_OUT] + p_ref[1][:, :D_OUT]) * t


def _deg_spec():
    return pl.BlockSpec((BLK, 1), lambda i: (i, 0))


def _tc_first(x, od, w):
    return pl.pallas_call(
        _tc_first_body,
        grid=(N_TAB // BLK,),
        in_specs=[
            pl.BlockSpec((BLK, D_IN), lambda i: (i, 0)),
            _deg_spec(),
            pl.BlockSpec((D_IN, D), lambda i: (0, 0)),
        ],
        out_specs=pl.BlockSpec((BLK, D), lambda i: (i, 0)),
        out_shape=jax.ShapeDtypeStruct((N_TAB, D), _f32),
    )(x, od, w)


def _tc_mid(p, od, idp, w):
    return pl.pallas_call(
        _tc_mid_body,
        grid=(N_TAB // BLK,),
        in_specs=[
            pl.BlockSpec((NC, BLK, D), lambda i: (0, i, 0)),
            _deg_spec(),
            _deg_spec(),
            pl.BlockSpec((D, D), lambda i: (0, 0)),
        ],
        out_specs=pl.BlockSpec((BLK, D), lambda i: (i, 0)),
        out_shape=jax.ShapeDtypeStruct((N_TAB, D), _f32),
    )(p, od, idp, w)


def _tc_last(p, idp):
    return pl.pallas_call(
        _tc_last_body,
        grid=(N_TAB // BLK,),
        in_specs=[
            pl.BlockSpec((NC, BLK, D), lambda i: (0, i, 0)),
            _deg_spec(),
        ],
        out_specs=pl.BlockSpec((BLK, D_OUT), lambda i: (i, 0)),
        out_shape=jax.ShapeDtypeStruct((N_TAB, D_OUT), _f32),
    )(p, idp)


def kernel(features, edge_index, W0, W1, W2):
    src = edge_index[0].astype(_i32)
    dst = edge_index[1].astype(_i32)
    src3 = jnp.pad(src, (0, E_PAD - E), constant_values=N).reshape(NW, CPT, CH)
    dst3 = jnp.pad(dst, (0, E_PAD - E), constant_values=N).reshape(NW, CPT, CH)
    x = jnp.pad(features, ((0, N_TAB - N), (0, 0)))
    w2p = jnp.pad(W2, ((0, 0), (0, D - D_OUT)))

    agg = _make_agg()
    odr, idr = _make_degrees()(src3, dst3)
    od = (odr[0, :, 0] + odr[1, :, 0])[:, None]
    idp = (idr[0, :, 0] + idr[1, :, 0])[:, None]

    y0 = _tc_first(x, od, W0)
    p0 = agg(y0, src3, dst3)
    y1 = _tc_mid(p0, od, idp, W1)
    p1 = agg(y1, src3, dst3)
    y2 = _tc_mid(p1, od, idp, w2p)
    p2 = agg(y2, src3, dst3)
    out = _tc_last(p2, idp)
    return out[:N]
